# baseline (speedup 1.0000x reference)
import functools

import jax
import jax.numpy as jnp
from jax import lax
from jax.experimental import pallas as pl
from jax.experimental.pallas import tpu as pltpu
from jax.experimental.pallas import tpu_sc as plsc

NUM_FIELDS = 23
VOCAB = 100000
EMBED_DIM = 64
BATCH = 4096

NC = 2
NS = 16
NW = NC * NS
NG = NUM_FIELDS * 8
VEC = 16
VC = 4096
WSHIFT = 12
NWIN = 25
LASTW = 1664
TAIL0 = (NWIN - 1) * VC + LASTW
TAILW = VOCAB - TAIL0
NWTOT = NWIN + 1
LSZ = BATCH + VEC


def _sc_gather_planes(inputs_t, tables_t, tables_tail):
    mesh = plsc.VectorSubcoreMesh(core_axis_name="c", subcore_axis_name="s")

    @functools.partial(
        pl.kernel,
        mesh=mesh,
        out_type=jax.ShapeDtypeStruct((NUM_FIELDS * EMBED_DIM, BATCH),
                                      jnp.float32),
        compiler_params=pltpu.CompilerParams(needs_layout_passes=False),
        scratch_types=[
            pltpu.VMEM((2, BATCH), jnp.int32),
            pltpu.VMEM((2 * LSZ,), jnp.int32),
            pltpu.VMEM((2 * LSZ,), jnp.int32),
            pltpu.SMEM((2 * (NWTOT + 1),), jnp.int32),
            pltpu.VMEM((1, 8, VC), jnp.float32),
            pltpu.VMEM((1, 8, VC), jnp.float32),
            pltpu.VMEM((1, 8, TAILW), jnp.float32),
            pltpu.VMEM((8, BATCH), jnp.float32),
            pltpu.SemaphoreType.DMA,
            pltpu.SemaphoreType.DMA,
        ],
    )
    def k(idx_hbm, tab_hbm, tail_hbm, out_hbm, idx_v, loc_v, pos_v, offs,
          buf0, buf1, tailbuf, rows, sem0, sem1):
        wid = lax.axis_index("s") * NC + lax.axis_index("c")
        g_lo = (NG * wid) // NW
        g_hi = (NG * (wid + 1)) // NW
        f0 = g_lo // 8
        f1 = jnp.minimum(f0 + 1, NUM_FIELDS - 1)
        pltpu.sync_copy(idx_hbm.at[pl.ds(f0, 1)], idx_v.at[pl.ds(0, 1)])
        pltpu.sync_copy(idx_hbm.at[pl.ds(f1, 1)], idx_v.at[pl.ds(1, 1)])
        offs[0] = 0
        offs[NWTOT + 1] = 0
        bufs = (buf0, buf1)
        sems = (sem0, sem1)
        iota = lax.iota(jnp.int32, VEC)
        zeros16 = jnp.zeros((VEC,), jnp.int32)

        def prep_window(fr, w, mode):
            frvec = zeros16 + fr
            obase = fr * (NWTOT + 1)
            lbase = fr * LSZ

            def body(i, off):
                c = i * VEC
                v = plsc.load_gather(idx_v, [frvec, iota + c])
                if mode == "tail":
                    m = v >= TAIL0
                    loc = v - TAIL0
                elif mode == "w24":
                    m = jnp.logical_and(
                        lax.shift_right_logical(v, WSHIFT) == w, v < TAIL0)
                    loc = v & (VC - 1)
                else:
                    m = lax.shift_right_logical(v, WSHIFT) == w
                    loc = v & (VC - 1)
                plsc.store_compressed(loc_v.at[pl.ds(lbase + off, VEC)],
                                      loc, mask=m)
                plsc.store_compressed(pos_v.at[pl.ds(lbase + off, VEC)],
                                      iota + c, mask=m)
                n = jnp.max(plsc.all_reduce_population_count(m))
                return off + n

            off_end = lax.fori_loop(0, BATCH // VEC, body, offs[obase + w])
            offs[obase + w + 1] = off_end

        def win_copy(g, w, par, width=VC):
            f = g // 8
            d0 = lax.rem(g, 8) * 8
            return pltpu.make_async_copy(
                tab_hbm.at[pl.ds(f, 1), pl.ds(d0, 8), pl.ds(w * VC, width)],
                bufs[par].at[:, :, pl.ds(0, width)], sems[par])

        def gather_window(src, fr, w):
            obase = fr * (NWTOT + 1)
            lbase = fr * LSZ
            start = offs[obase + w]
            end = offs[obase + w + 1]
            rem = lax.rem(end - start, VEC)
            full_end = end - rem
            for d in range(8):
                dvec = zeros16 + d

                @pl.loop(start, full_end, step=VEC)
                def _(c):
                    loc = loc_v.at[pl.ds(lbase + c, VEC)][...]
                    pos = pos_v.at[pl.ds(lbase + c, VEC)][...]
                    plsc.store_scatter(
                        rows, [dvec, pos],
                        plsc.load_gather(src, [zeros16, dvec, loc]))

                @pl.when(rem > 0)
                def _():
                    m = iota < rem
                    loc = loc_v.at[pl.ds(lbase + full_end, VEC)][...]
                    pos = pos_v.at[pl.ds(lbase + full_end, VEC)][...]
                    plsc.store_scatter(
                        rows, [dvec, pos],
                        plsc.load_gather(src, [zeros16, dvec, loc],
                                         mask=m), mask=m)

        @pl.loop(g_lo, g_hi)
        def _(g):
            f = g // 8
            fr = f - f0
            d0 = lax.rem(g, 8) * 8
            first = jnp.logical_or(g == g_lo, lax.rem(g, 8) == 0)
            win_copy(g, 0, 0).start()
            pltpu.sync_copy(tail_hbm.at[pl.ds(f, 1), pl.ds(d0, 8)], tailbuf)

            @pl.loop(0, NWIN - 1, step=2)
            def _(j):
                for par in (0, 1):
                    w = j + par

                    @pl.when(first)
                    def _():
                        prep_window(fr, w, "main")

                    win_copy(g, w, par).wait()

                    @pl.when(w + 1 < NWIN - 1)
                    def _():
                        win_copy(g, w + 1, 1 - par).start()

                    gather_window(bufs[par], fr, w)

            win_copy(g, NWIN - 1, 0, LASTW).start()

            @pl.when(first)
            def _():
                prep_window(fr, NWIN - 1, "w24")
                prep_window(fr, NWIN, "tail")

            win_copy(g, NWIN - 1, 0, LASTW).wait()
            gather_window(bufs[0], fr, NWIN - 1)
            gather_window(tailbuf, fr, NWIN)

            for d in range(8):
                pltpu.sync_copy(rows.at[pl.ds(d, 1)],
                                out_hbm.at[pl.ds(g * 8 + d, 1)])

    return k(inputs_t, tables_t, tables_tail)


def kernel(inputs, tables):
    inputs_t = inputs.T
    tables_t = jnp.transpose(tables, (0, 2, 1))
    tables_tail = tables_t[:, :, TAIL0:]
    out_t = _sc_gather_planes(inputs_t, tables_t, tables_tail)
    return out_t.T


# speedup vs baseline: 1.2540x; 1.2540x over previous
import functools

import jax
import jax.numpy as jnp
from jax import lax
from jax.experimental import pallas as pl
from jax.experimental.pallas import tpu as pltpu
from jax.experimental.pallas import tpu_sc as plsc

NUM_FIELDS = 23
VOCAB = 100000
EMBED_DIM = 64
BATCH = 4096

NC = 2
NS = 16
NW = NC * NS
NPLANES = NUM_FIELDS * EMBED_DIM
PPW = NPLANES // NW
VEC = 16


def _sc_gather_planes(inputs_t, tables_t):
    mesh = plsc.VectorSubcoreMesh(core_axis_name="c", subcore_axis_name="s")

    @functools.partial(
        pl.kernel,
        mesh=mesh,
        out_type=jax.ShapeDtypeStruct((NPLANES, BATCH), jnp.float32),
        compiler_params=pltpu.CompilerParams(needs_layout_passes=False),
        scratch_types=[
            pltpu.VMEM((2, BATCH), jnp.int32),
            pltpu.VMEM((VOCAB,), jnp.float32),
            pltpu.VMEM((BATCH,), jnp.float32),
        ],
    )
    def k(idx_hbm, tab_hbm, out_hbm, idx_v, plane_v, row_v):
        wid = lax.axis_index("s") * NC + lax.axis_index("c")
        p0 = wid * PPW
        f0 = p0 // EMBED_DIM
        f1 = jnp.minimum(f0 + 1, NUM_FIELDS - 1)
        pltpu.sync_copy(idx_hbm.at[pl.ds(f0, 1)], idx_v.at[pl.ds(0, 1)])
        pltpu.sync_copy(idx_hbm.at[pl.ds(f1, 1)], idx_v.at[pl.ds(1, 1)])
        iota = lax.iota(jnp.int32, VEC)
        zeros16 = jnp.zeros((VEC,), jnp.int32)

        @pl.loop(0, PPW)
        def _(j):
            p = p0 + j
            f = p // EMBED_DIM
            d = lax.rem(p, EMBED_DIM)
            frvec = zeros16 + (f - f0)
            pltpu.sync_copy(tab_hbm.at[f, d], plane_v)

            @pl.loop(0, BATCH, step=2 * VEC)
            def _(c):
                for u in range(2):
                    cc = c + u * VEC
                    idx = plsc.load_gather(idx_v, [frvec, iota + cc])
                    row_v.at[pl.ds(cc, VEC)][...] = plsc.load_gather(
                        plane_v, [idx])

            pltpu.sync_copy(row_v, out_hbm.at[p])

    return k(inputs_t, tables_t)


def kernel(inputs, tables):
    inputs_t = inputs.T
    tables_t = jnp.transpose(tables, (0, 2, 1))
    out_t = _sc_gather_planes(inputs_t, tables_t)
    return out_t.T
